# baseline (speedup 1.0000x reference)
import jax
import jax.numpy as jnp
from jax.experimental import pallas as pl
from jax.experimental.pallas import tpu as pltpu

B = 8
D_MODEL = 1024
VOCAB = 100000
NS = 2
BVS = 2048
STEP = NS * BVS
_LAST_BLK = (VOCAB - 1) // BVS
NEG = -1e30


def _body(msg_ref, wt0_ref, wt1_ref, b_ref, ns_ref, lp_ref, ent_ref,
          m_ref, s_ref, t_ref, idx_ref):
    i = pl.program_id(0)
    nb = pl.num_programs(0)

    @pl.when(i == 0)
    def _init():
        m_ref[...] = jnp.full((B, 1), NEG, jnp.float32)
        s_ref[...] = jnp.zeros((B, 1), jnp.float32)
        t_ref[...] = jnp.zeros((B, 1), jnp.float32)
        idx_ref[...] = jnp.zeros((B, 1), jnp.int32)

    for k, wt_ref in enumerate((wt0_ref, wt1_ref)):
        logits = jax.lax.dot_general(
            msg_ref[...], wt_ref[...], (((1,), (1,)), ((), ())),
            preferred_element_type=jnp.float32)
        base = i * STEP + k * BVS
        logits = logits + b_ref[:, k * BVS:(k + 1) * BVS]
        col = base + jax.lax.broadcasted_iota(jnp.int32, (B, BVS), 1)
        logits = jnp.where(col < VOCAB, logits, NEG)

        bmax = jnp.max(logits, axis=1, keepdims=True)
        cand = jnp.where(logits == bmax, col, jnp.int32(2**31 - 1))
        bidx = jnp.min(cand, axis=1, keepdims=True)

        m_old = m_ref[...]
        new_m = jnp.maximum(m_old, bmax)
        e = jnp.exp(logits - new_m)
        scale = jnp.exp(m_old - new_m)
        s_ref[...] = s_ref[...] * scale + jnp.sum(e, axis=1, keepdims=True)
        t_ref[...] = (t_ref[...] * scale
                      + jnp.sum(logits * e, axis=1, keepdims=True))
        m_ref[...] = new_m
        idx_ref[...] = jnp.where(bmax > m_old, bidx, idx_ref[...])

    @pl.when(i == nb - 1)
    def _fin():
        m = m_ref[...]
        s = s_ref[...]
        lse = m + jnp.log(s)
        ns_ref[...] = idx_ref[...]
        lp_ref[...] = m - lse
        ent_ref[...] = lse - t_ref[...] / s


@jax.jit
def kernel(message, W, b):
    nb = pl.cdiv(VOCAB, STEP)
    wt = W.T
    b2 = b.reshape(1, VOCAB)
    ns, lp, ent = pl.pallas_call(
        _body,
        grid=(nb,),
        in_specs=[
            pl.BlockSpec((B, D_MODEL), lambda i: (0, 0)),
            pl.BlockSpec((BVS, D_MODEL),
                         lambda i: (jnp.minimum(2 * i, _LAST_BLK), 0)),
            pl.BlockSpec((BVS, D_MODEL),
                         lambda i: (jnp.minimum(2 * i + 1, _LAST_BLK), 0)),
            pl.BlockSpec((1, STEP), lambda i: (0, i)),
        ],
        out_specs=[
            pl.BlockSpec((B, 1), lambda i: (0, 0)),
            pl.BlockSpec((B, 1), lambda i: (0, 0)),
            pl.BlockSpec((B, 1), lambda i: (0, 0)),
        ],
        out_shape=[
            jax.ShapeDtypeStruct((B, 1), jnp.int32),
            jax.ShapeDtypeStruct((B, 1), jnp.float32),
            jax.ShapeDtypeStruct((B, 1), jnp.float32),
        ],
        scratch_shapes=[
            pltpu.VMEM((B, 1), jnp.float32),
            pltpu.VMEM((B, 1), jnp.float32),
            pltpu.VMEM((B, 1), jnp.float32),
            pltpu.VMEM((B, 1), jnp.int32),
        ],
    )(message, wt, wt, b2)
    return ns[:, 0], lp[:, 0], ent[:, 0]
